# baseline (speedup 1.0000x reference)
import functools

import jax
import jax.numpy as jnp
from jax import lax
from jax.experimental import pallas as pl
from jax.experimental.pallas import tpu as pltpu

_NUM_CLASSES = 21
_KNN = 5
_CONCENTRATION = 16.0
_LAB_BITS = 5


def _topk_vote(src, key, bq, o_ref):
    big = jnp.float32(3e38)
    neg = jnp.float32(-jnp.inf)
    cls = lax.broadcasted_iota(jnp.int32, (1, 32), 1)
    scores = jnp.zeros((bq, 32), jnp.float32)
    work = src[...]
    for it in range(_KNN):
        m = jnp.max(work, axis=1, keepdims=True)
        amin = jnp.min(jnp.where(work == m, key, big),
                       axis=1, keepdims=True)
        lab = amin.astype(jnp.int32) & (2 ** _LAB_BITS - 1)
        scores = scores + m * (lab == cls).astype(jnp.float32)
        if it < _KNN - 1:
            src[...] = jnp.where(key == amin, neg, work)
            work = src[...]
    o_ref[...] = scores[:, :_NUM_CLASSES]


def _pnorm_body(p_ref, pn_ref):
    p = p_ref[...]
    pnorm = jnp.sqrt(jnp.sum(p * p, axis=1, keepdims=True)) + 1e-12
    pn_ref[...] = p / pnorm


def _body(e_ref, pn_ref, lab_ref, o_ref, sa_ref, sb_ref, *, bq, k, d, nqb):
    i = pl.program_id(0)

    def matmul(dst):
        e = e_ref[...]
        scale = _CONCENTRATION / (
            jnp.sqrt(jnp.sum(e * e, axis=1, keepdims=True)) + 1e-12)
        dst[...] = lax.dot_general(
            e * scale, pn_ref[...], (((1,), (1,)), ((), ())),
            preferred_element_type=jnp.float32,
        )

    def vote(src):
        col = lax.broadcasted_iota(jnp.int32, (1, k), 1)
        key = ((col << _LAB_BITS) | lab_ref[...]).astype(jnp.float32)
        _topk_vote(src, key, bq, o_ref)

    @pl.when(i % 2 == 0)
    def _():
        @pl.when(i < nqb)
        def _():
            matmul(sa_ref)

        @pl.when(i > 0)
        def _():
            vote(sb_ref)

    @pl.when(i % 2 == 1)
    def _():
        @pl.when(i < nqb)
        def _():
            matmul(sb_ref)

        vote(sa_ref)


def kernel(embeddings, prototypes, prototype_labels):
    q, d = embeddings.shape
    k = prototypes.shape[0]
    bq = 256
    nqb = q // bq
    labels2d = prototype_labels.reshape(1, k)

    pn = pl.pallas_call(
        _pnorm_body,
        out_shape=jax.ShapeDtypeStruct((k, d), jnp.float32),
    )(prototypes)

    return pl.pallas_call(
        functools.partial(_body, bq=bq, k=k, d=d, nqb=nqb),
        grid=(nqb + 1,),
        in_specs=[
            pl.BlockSpec((bq, d), lambda i: (jnp.minimum(i, nqb - 1), 0)),
            pl.BlockSpec((k, d), lambda i: (0, 0)),
            pl.BlockSpec((1, k), lambda i: (0, 0)),
        ],
        out_specs=pl.BlockSpec(
            (bq, _NUM_CLASSES), lambda i: (jnp.maximum(i - 1, 0), 0)),
        out_shape=jax.ShapeDtypeStruct((q, _NUM_CLASSES), jnp.float32),
        scratch_shapes=[
            pltpu.VMEM((bq, k), jnp.float32),
            pltpu.VMEM((bq, k), jnp.float32),
        ],
    )(embeddings, pn, labels2d)


# speedup vs baseline: 2.4639x; 2.4639x over previous
import functools

import jax
import jax.numpy as jnp
from jax import lax
from jax.experimental import pallas as pl
from jax.experimental.pallas import tpu as pltpu

_NUM_CLASSES = 21
_KNN = 5
_CONCENTRATION = 16.0
_LAB_BITS = 5


def _topk_vote(sim, key, bq):
    big = jnp.float32(3e38)
    neg = jnp.float32(-jnp.inf)
    cls = lax.broadcasted_iota(jnp.int32, (1, 32), 1)
    scores = jnp.zeros((bq, 32), jnp.float32)
    work = sim
    for it in range(_KNN):
        m = jnp.max(work, axis=1, keepdims=True)
        amin = jnp.min(jnp.where(work == m, key, big),
                       axis=1, keepdims=True)
        lab = amin.astype(jnp.int32) & (2 ** _LAB_BITS - 1)
        scores = scores + m * (lab == cls).astype(jnp.float32)
        if it < _KNN - 1:
            work = jnp.where(key == amin, neg, work)
    return scores


def _body(e_ref, p_ref, lab_ref, o_ref, pn_ref, *, bq, k, d):
    @pl.when(pl.program_id(0) == 0)
    def _():
        p = p_ref[...]
        pnorm = jnp.sqrt(jnp.sum(p * p, axis=1, keepdims=True)) + 1e-12
        pn_ref[...] = p / pnorm

    e = e_ref[...]
    scale = _CONCENTRATION / (
        jnp.sqrt(jnp.sum(e * e, axis=1, keepdims=True)) + 1e-12)
    sim = lax.dot_general(
        e * scale, pn_ref[...], (((1,), (1,)), ((), ())),
        preferred_element_type=jnp.float32,
    )

    col = lax.broadcasted_iota(jnp.int32, (1, k), 1)
    key = ((col << _LAB_BITS) | lab_ref[...]).astype(jnp.float32)
    scores = _topk_vote(sim, key, bq)
    o_ref[...] = scores[:, :_NUM_CLASSES]


def kernel(embeddings, prototypes, prototype_labels):
    q, d = embeddings.shape
    k = prototypes.shape[0]
    bq = 256
    labels2d = prototype_labels.reshape(1, k)

    return pl.pallas_call(
        functools.partial(_body, bq=bq, k=k, d=d),
        grid=(q // bq,),
        in_specs=[
            pl.BlockSpec((bq, d), lambda i: (i, 0)),
            pl.BlockSpec((k, d), lambda i: (0, 0)),
            pl.BlockSpec((1, k), lambda i: (0, 0)),
        ],
        out_specs=pl.BlockSpec((bq, _NUM_CLASSES), lambda i: (i, 0)),
        out_shape=jax.ShapeDtypeStruct((q, _NUM_CLASSES), jnp.float32),
        scratch_shapes=[pltpu.VMEM((k, d), jnp.float32)],
    )(embeddings, prototypes, labels2d)
